# Initial kernel scaffold; baseline (speedup 1.0000x reference)
import jax
import jax.numpy as jnp
from jax.experimental import pallas as pl


def kernel(h_sc, h_st, bi_e, bi_graph, sc_e, sc_graph, st_e, st_graph, params):
    raise NotImplementedError("write your pallas kernel here")


# speedup vs baseline: 2.3815x; 2.3815x over previous
import jax
import jax.numpy as jnp
from jax.experimental import pallas as pl
from jax.experimental.pallas import tpu as pltpu

_EPS = 1e-5
_BI = 8


def _proj_kernel(x_ref, w_ref, b_ref, o_ref):
    x = x_ref[:]
    for k in range(w_ref.shape[0]):
        o_ref[k] = (jnp.dot(x, w_ref[k], preferred_element_type=jnp.float32)
                    + b_ref[k][None, :])


def _edge_body(e_ref, a_ref, bc_ref, mrow_ref, mcol_ref, cw_ref, cb_ref,
               ge_ref, be_ref, eo_ref, row_ref, col_ref):
    x = e_ref[0]
    xc = jax.lax.dot_general(x, cw_ref[:], (((2,), (0,)), ((), ())),
                             preferred_element_type=jnp.float32)
    e_new = (xc + a_ref[0][:, None, :] + bc_ref[0][None, :, :]
             + cb_ref[:].reshape(1, 1, -1))
    g = jax.nn.sigmoid(e_new)
    row_ref[0] = jnp.sum(g * mrow_ref[0][None, :, :], axis=1)
    if col_ref is not None:
        colc = jnp.sum(g * mcol_ref[0][:, None, :], axis=0)

        @pl.when(pl.program_id(1) == 0)
        def _():
            col_ref[0] = colc

        @pl.when(pl.program_id(1) > 0)
        def _():
            col_ref[0] = col_ref[0] + colc

    mu = jnp.mean(e_new, axis=-1, keepdims=True)
    var = jnp.mean((e_new - mu) ** 2, axis=-1, keepdims=True)
    ln = ((e_new - mu) * jax.lax.rsqrt(var + _EPS)
          * ge_ref[:].reshape(1, 1, -1) + be_ref[:].reshape(1, 1, -1))
    eo_ref[0] = x + jnp.maximum(ln, 0.0)


def _edge_kernel_col(e, a, bc, mr, mc, cw, cb, ge, be, eo, row, col):
    _edge_body(e, a, bc, mr, mc, cw, cb, ge, be, eo, row, col)


def _edge_kernel_nocol(e, a, bc, mr, cw, cb, ge, be, eo, row):
    _edge_body(e, a, bc, mr, None, cw, cb, ge, be, eo, row, None)


def _edge_pass(e, a, bc, mrow, mcol, cw, cb, ge, be):
    Bb, Vi, Vj, Hh = e.shape
    grid = (Bb, Vi // _BI)
    in_specs = [
        pl.BlockSpec((1, _BI, Vj, Hh), lambda b, i: (b, i, 0, 0)),
        pl.BlockSpec((1, _BI, Hh), lambda b, i: (b, i, 0)),
        pl.BlockSpec((1, Vj, Hh), lambda b, i: (b, 0, 0)),
        pl.BlockSpec((1, Vj, Hh), lambda b, i: (b, 0, 0)),
    ]
    if mcol is not None:
        in_specs.append(pl.BlockSpec((1, _BI, Hh), lambda b, i: (b, i, 0)))
    in_specs += [
        pl.BlockSpec((Hh, Hh), lambda b, i: (0, 0)),
        pl.BlockSpec((1, Hh), lambda b, i: (0, 0)),
        pl.BlockSpec((1, Hh), lambda b, i: (0, 0)),
        pl.BlockSpec((1, Hh), lambda b, i: (0, 0)),
    ]
    out_shape = [jax.ShapeDtypeStruct(e.shape, e.dtype),
                 jax.ShapeDtypeStruct((Bb, Vi, Hh), e.dtype)]
    out_specs = [pl.BlockSpec((1, _BI, Vj, Hh), lambda b, i: (b, i, 0, 0)),
                 pl.BlockSpec((1, _BI, Hh), lambda b, i: (b, i, 0))]
    if mcol is not None:
        out_shape.append(jax.ShapeDtypeStruct((Bb, Vj, Hh), e.dtype))
        out_specs.append(pl.BlockSpec((1, Vj, Hh), lambda b, i: (b, 0, 0)))
    fn = _edge_kernel_col if mcol is not None else _edge_kernel_nocol
    args = (e, a, bc, mrow) + ((mcol,) if mcol is not None else ()) + (
        cw, cb, ge, be)
    return pl.pallas_call(
        fn, grid=grid, in_specs=in_specs, out_specs=out_specs,
        out_shape=out_shape,
        compiler_params=pltpu.CompilerParams(
            dimension_semantics=("arbitrary", "arbitrary")),
    )(*args)


def _node_kernel(h_ref, uh_ref, a1_ref, a2_ref, g_ref, b_ref, o_ref):
    s = uh_ref[:] + a1_ref[:] + a2_ref[:]
    mu = jnp.mean(s, axis=-1, keepdims=True)
    var = jnp.mean((s - mu) ** 2, axis=-1, keepdims=True)
    ln = (s - mu) * jax.lax.rsqrt(var + _EPS) * g_ref[:] + b_ref[:]
    o_ref[:] = h_ref[:] + jnp.maximum(ln, 0.0)


def _node_pass(h, uh, a1, a2, g, b):
    n, Hh = h.shape
    return pl.pallas_call(
        _node_kernel,
        out_shape=jax.ShapeDtypeStruct((n, Hh), h.dtype),
    )(h, uh, a1, a2, g, b)


def kernel(h_sc, h_st, bi_e, bi_graph, sc_e, sc_graph, st_e, st_graph, params):
    Bb, Vsc, Hh = h_sc.shape
    Vst = h_st.shape[1]
    p = params

    names_sc = ["U1", "V1", "W1", "bi_A", "sc_A", "sc_B"]
    names_st = ["U2", "V2", "W2", "bi_B", "st_A", "st_B"]
    w_sc = jnp.stack([p[n]["w"].T for n in names_sc])
    b_sc = jnp.stack([p[n]["b"] for n in names_sc])
    w_st = jnp.stack([p[n]["w"].T for n in names_st])
    b_st = jnp.stack([p[n]["b"] for n in names_st])

    def proj(x, w, b):
        n = x.shape[0] * x.shape[1]
        return pl.pallas_call(
            _proj_kernel,
            out_shape=jax.ShapeDtypeStruct((w.shape[0], n, Hh), jnp.float32),
        )(x.reshape(n, Hh), w, b)

    proj_sc = proj(h_sc, w_sc, b_sc)
    proj_st = proj(h_st, w_st, b_st)
    Uh_sc, Vh_sc, Wh_sc, Abi, Asc, Bsc = (
        proj_sc[k].reshape(Bb, Vsc, Hh) for k in range(6))
    Uh_st, Vh_st, Wh_st, Bbi, Ast, Bst = (
        proj_st[k].reshape(Bb, Vst, Hh) for k in range(6))

    ge = p["ln_e"]["g"][None, :]
    be = p["ln_e"]["b"][None, :]
    gh = p["ln_h"]["g"][None, :]
    bh = p["ln_h"]["b"][None, :]

    bi_out, st2sc, sc2st = _edge_pass(
        bi_e, Abi, Bbi, Vh_st, Vh_sc,
        p["bi_C"]["w"].T, p["bi_C"]["b"][None, :], ge, be)
    sc_out, sc2sc = _edge_pass(
        sc_e, Asc, Bsc, Wh_sc, None,
        p["sc_C"]["w"].T, p["sc_C"]["b"][None, :], ge, be)
    st_out, st2st = _edge_pass(
        st_e, Ast, Bst, Wh_st, None,
        p["st_C"]["w"].T, p["st_C"]["b"][None, :], ge, be)

    h_sc_out = _node_pass(
        h_sc.reshape(Bb * Vsc, Hh), Uh_sc.reshape(Bb * Vsc, Hh),
        st2sc.reshape(Bb * Vsc, Hh), sc2sc.reshape(Bb * Vsc, Hh),
        gh, bh).reshape(Bb, Vsc, Hh)
    h_st_out = _node_pass(
        h_st.reshape(Bb * Vst, Hh), Uh_st.reshape(Bb * Vst, Hh),
        sc2st.reshape(Bb * Vst, Hh), st2st.reshape(Bb * Vst, Hh),
        gh, bh).reshape(Bb, Vst, Hh)

    return (h_sc_out, h_st_out, bi_out, sc_out, st_out)


# speedup vs baseline: 2.9362x; 1.2330x over previous
import jax
import jax.numpy as jnp
from jax.experimental import pallas as pl
from jax.experimental.pallas import tpu as pltpu

_EPS = 1e-5
_BI = 16


def _proj_kernel(x_ref, w_ref, b_ref, o_ref):
    x = x_ref[:]
    for k in range(w_ref.shape[0]):
        o_ref[k] = (jnp.dot(x, w_ref[k], preferred_element_type=jnp.float32)
                    + b_ref[k][None, :])


def _edge_body(e_ref, a_ref, bc_ref, mrow_ref, mcol_ref, cw_ref, cb_ref,
               ge_ref, be_ref, eo_ref, row_ref, col_ref):
    x = e_ref[0]
    xc = jax.lax.dot_general(x, cw_ref[:], (((2,), (0,)), ((), ())),
                             preferred_element_type=jnp.float32)
    e_new = (xc + a_ref[0][:, None, :] + bc_ref[0][None, :, :]
             + cb_ref[:].reshape(1, 1, -1))
    g = jax.nn.sigmoid(e_new)
    row_ref[0] = jnp.sum(g * mrow_ref[0][None, :, :], axis=1)
    if col_ref is not None:
        colc = jnp.sum(g * mcol_ref[0][:, None, :], axis=0)

        @pl.when(pl.program_id(1) == 0)
        def _():
            col_ref[0] = colc

        @pl.when(pl.program_id(1) > 0)
        def _():
            col_ref[0] = col_ref[0] + colc

    mu = jnp.mean(e_new, axis=-1, keepdims=True)
    var = jnp.mean((e_new - mu) ** 2, axis=-1, keepdims=True)
    ln = ((e_new - mu) * jax.lax.rsqrt(var + _EPS)
          * ge_ref[:].reshape(1, 1, -1) + be_ref[:].reshape(1, 1, -1))
    eo_ref[0] = x + jnp.maximum(ln, 0.0)


def _edge_kernel_col(e, a, bc, mr, mc, cw, cb, ge, be, eo, row, col):
    _edge_body(e, a, bc, mr, mc, cw, cb, ge, be, eo, row, col)


def _edge_kernel_nocol(e, a, bc, mr, cw, cb, ge, be, eo, row):
    _edge_body(e, a, bc, mr, None, cw, cb, ge, be, eo, row, None)


def _edge_pass(e, a, bc, mrow, mcol, cw, cb, ge, be):
    Bb, Vi, Vj, Hh = e.shape
    grid = (Bb, Vi // _BI)
    in_specs = [
        pl.BlockSpec((1, _BI, Vj, Hh), lambda b, i: (b, i, 0, 0)),
        pl.BlockSpec((1, _BI, Hh), lambda b, i: (b, i, 0)),
        pl.BlockSpec((1, Vj, Hh), lambda b, i: (b, 0, 0)),
        pl.BlockSpec((1, Vj, Hh), lambda b, i: (b, 0, 0)),
    ]
    if mcol is not None:
        in_specs.append(pl.BlockSpec((1, _BI, Hh), lambda b, i: (b, i, 0)))
    in_specs += [
        pl.BlockSpec((Hh, Hh), lambda b, i: (0, 0)),
        pl.BlockSpec((1, Hh), lambda b, i: (0, 0)),
        pl.BlockSpec((1, Hh), lambda b, i: (0, 0)),
        pl.BlockSpec((1, Hh), lambda b, i: (0, 0)),
    ]
    out_shape = [jax.ShapeDtypeStruct(e.shape, e.dtype),
                 jax.ShapeDtypeStruct((Bb, Vi, Hh), e.dtype)]
    out_specs = [pl.BlockSpec((1, _BI, Vj, Hh), lambda b, i: (b, i, 0, 0)),
                 pl.BlockSpec((1, _BI, Hh), lambda b, i: (b, i, 0))]
    if mcol is not None:
        out_shape.append(jax.ShapeDtypeStruct((Bb, Vj, Hh), e.dtype))
        out_specs.append(pl.BlockSpec((1, Vj, Hh), lambda b, i: (b, 0, 0)))
    fn = _edge_kernel_col if mcol is not None else _edge_kernel_nocol
    args = (e, a, bc, mrow) + ((mcol,) if mcol is not None else ()) + (
        cw, cb, ge, be)
    return pl.pallas_call(
        fn, grid=grid, in_specs=in_specs, out_specs=out_specs,
        out_shape=out_shape,
        compiler_params=pltpu.CompilerParams(
            dimension_semantics=("arbitrary", "arbitrary")),
    )(*args)


def _node_kernel(h_ref, uh_ref, a1_ref, a2_ref, g_ref, b_ref, o_ref):
    s = uh_ref[:] + a1_ref[:] + a2_ref[:]
    mu = jnp.mean(s, axis=-1, keepdims=True)
    var = jnp.mean((s - mu) ** 2, axis=-1, keepdims=True)
    ln = (s - mu) * jax.lax.rsqrt(var + _EPS) * g_ref[:] + b_ref[:]
    o_ref[:] = h_ref[:] + jnp.maximum(ln, 0.0)


def _node_pass(h, uh, a1, a2, g, b):
    n, Hh = h.shape
    return pl.pallas_call(
        _node_kernel,
        out_shape=jax.ShapeDtypeStruct((n, Hh), h.dtype),
    )(h, uh, a1, a2, g, b)


def kernel(h_sc, h_st, bi_e, bi_graph, sc_e, sc_graph, st_e, st_graph, params):
    Bb, Vsc, Hh = h_sc.shape
    Vst = h_st.shape[1]
    p = params

    names_sc = ["U1", "V1", "W1", "bi_A", "sc_A", "sc_B"]
    names_st = ["U2", "V2", "W2", "bi_B", "st_A", "st_B"]
    w_sc = jnp.stack([p[n]["w"].T for n in names_sc])
    b_sc = jnp.stack([p[n]["b"] for n in names_sc])
    w_st = jnp.stack([p[n]["w"].T for n in names_st])
    b_st = jnp.stack([p[n]["b"] for n in names_st])

    def proj(x, w, b):
        n = x.shape[0] * x.shape[1]
        return pl.pallas_call(
            _proj_kernel,
            out_shape=jax.ShapeDtypeStruct((w.shape[0], n, Hh), jnp.float32),
        )(x.reshape(n, Hh), w, b)

    proj_sc = proj(h_sc, w_sc, b_sc)
    proj_st = proj(h_st, w_st, b_st)
    Uh_sc, Vh_sc, Wh_sc, Abi, Asc, Bsc = (
        proj_sc[k].reshape(Bb, Vsc, Hh) for k in range(6))
    Uh_st, Vh_st, Wh_st, Bbi, Ast, Bst = (
        proj_st[k].reshape(Bb, Vst, Hh) for k in range(6))

    ge = p["ln_e"]["g"][None, :]
    be = p["ln_e"]["b"][None, :]
    gh = p["ln_h"]["g"][None, :]
    bh = p["ln_h"]["b"][None, :]

    bi_out, st2sc, sc2st = _edge_pass(
        bi_e, Abi, Bbi, Vh_st, Vh_sc,
        p["bi_C"]["w"].T, p["bi_C"]["b"][None, :], ge, be)
    sc_out, sc2sc = _edge_pass(
        sc_e, Asc, Bsc, Wh_sc, None,
        p["sc_C"]["w"].T, p["sc_C"]["b"][None, :], ge, be)
    st_out, st2st = _edge_pass(
        st_e, Ast, Bst, Wh_st, None,
        p["st_C"]["w"].T, p["st_C"]["b"][None, :], ge, be)

    h_sc_out = _node_pass(
        h_sc.reshape(Bb * Vsc, Hh), Uh_sc.reshape(Bb * Vsc, Hh),
        st2sc.reshape(Bb * Vsc, Hh), sc2sc.reshape(Bb * Vsc, Hh),
        gh, bh).reshape(Bb, Vsc, Hh)
    h_st_out = _node_pass(
        h_st.reshape(Bb * Vst, Hh), Uh_st.reshape(Bb * Vst, Hh),
        sc2st.reshape(Bb * Vst, Hh), st2st.reshape(Bb * Vst, Hh),
        gh, bh).reshape(Bb, Vst, Hh)

    return (h_sc_out, h_st_out, bi_out, sc_out, st_out)


# speedup vs baseline: 3.2363x; 1.1022x over previous
import jax
import jax.numpy as jnp
from jax.experimental import pallas as pl
from jax.experimental.pallas import tpu as pltpu

_EPS = 1e-5
_BI = 32


def _proj_kernel(x_ref, w_ref, b_ref, o_ref):
    x = x_ref[:]
    for k in range(w_ref.shape[0]):
        o_ref[k] = (jnp.dot(x, w_ref[k], preferred_element_type=jnp.float32)
                    + b_ref[k][None, :])


def _edge_body(e_ref, a_ref, bc_ref, mrow_ref, mcol_ref, cw_ref, cb_ref,
               ge_ref, be_ref, eo_ref, row_ref, col_ref):
    x = e_ref[0]
    xc = jax.lax.dot_general(x, cw_ref[:], (((2,), (0,)), ((), ())),
                             preferred_element_type=jnp.float32)
    e_new = (xc + a_ref[0][:, None, :] + bc_ref[0][None, :, :]
             + cb_ref[:].reshape(1, 1, -1))
    g = jax.nn.sigmoid(e_new)
    row_ref[0] = jnp.sum(g * mrow_ref[0][None, :, :], axis=1)
    if col_ref is not None:
        colc = jnp.sum(g * mcol_ref[0][:, None, :], axis=0)

        @pl.when(pl.program_id(1) == 0)
        def _():
            col_ref[0] = colc

        @pl.when(pl.program_id(1) > 0)
        def _():
            col_ref[0] = col_ref[0] + colc

    mu = jnp.mean(e_new, axis=-1, keepdims=True)
    var = jnp.mean((e_new - mu) ** 2, axis=-1, keepdims=True)
    ln = ((e_new - mu) * jax.lax.rsqrt(var + _EPS)
          * ge_ref[:].reshape(1, 1, -1) + be_ref[:].reshape(1, 1, -1))
    eo_ref[0] = x + jnp.maximum(ln, 0.0)


def _edge_kernel_col(e, a, bc, mr, mc, cw, cb, ge, be, eo, row, col):
    _edge_body(e, a, bc, mr, mc, cw, cb, ge, be, eo, row, col)


def _edge_kernel_nocol(e, a, bc, mr, cw, cb, ge, be, eo, row):
    _edge_body(e, a, bc, mr, None, cw, cb, ge, be, eo, row, None)


def _edge_pass(e, a, bc, mrow, mcol, cw, cb, ge, be):
    Bb, Vi, Vj, Hh = e.shape
    grid = (Bb, Vi // _BI)
    in_specs = [
        pl.BlockSpec((1, _BI, Vj, Hh), lambda b, i: (b, i, 0, 0)),
        pl.BlockSpec((1, _BI, Hh), lambda b, i: (b, i, 0)),
        pl.BlockSpec((1, Vj, Hh), lambda b, i: (b, 0, 0)),
        pl.BlockSpec((1, Vj, Hh), lambda b, i: (b, 0, 0)),
    ]
    if mcol is not None:
        in_specs.append(pl.BlockSpec((1, _BI, Hh), lambda b, i: (b, i, 0)))
    in_specs += [
        pl.BlockSpec((Hh, Hh), lambda b, i: (0, 0)),
        pl.BlockSpec((1, Hh), lambda b, i: (0, 0)),
        pl.BlockSpec((1, Hh), lambda b, i: (0, 0)),
        pl.BlockSpec((1, Hh), lambda b, i: (0, 0)),
    ]
    out_shape = [jax.ShapeDtypeStruct(e.shape, e.dtype),
                 jax.ShapeDtypeStruct((Bb, Vi, Hh), e.dtype)]
    out_specs = [pl.BlockSpec((1, _BI, Vj, Hh), lambda b, i: (b, i, 0, 0)),
                 pl.BlockSpec((1, _BI, Hh), lambda b, i: (b, i, 0))]
    if mcol is not None:
        out_shape.append(jax.ShapeDtypeStruct((Bb, Vj, Hh), e.dtype))
        out_specs.append(pl.BlockSpec((1, Vj, Hh), lambda b, i: (b, 0, 0)))
    fn = _edge_kernel_col if mcol is not None else _edge_kernel_nocol
    args = (e, a, bc, mrow) + ((mcol,) if mcol is not None else ()) + (
        cw, cb, ge, be)
    return pl.pallas_call(
        fn, grid=grid, in_specs=in_specs, out_specs=out_specs,
        out_shape=out_shape,
        compiler_params=pltpu.CompilerParams(
            dimension_semantics=("arbitrary", "arbitrary")),
    )(*args)


def _node_kernel(h_ref, uh_ref, a1_ref, a2_ref, g_ref, b_ref, o_ref):
    s = uh_ref[:] + a1_ref[:] + a2_ref[:]
    mu = jnp.mean(s, axis=-1, keepdims=True)
    var = jnp.mean((s - mu) ** 2, axis=-1, keepdims=True)
    ln = (s - mu) * jax.lax.rsqrt(var + _EPS) * g_ref[:] + b_ref[:]
    o_ref[:] = h_ref[:] + jnp.maximum(ln, 0.0)


def _node_pass(h, uh, a1, a2, g, b):
    n, Hh = h.shape
    return pl.pallas_call(
        _node_kernel,
        out_shape=jax.ShapeDtypeStruct((n, Hh), h.dtype),
    )(h, uh, a1, a2, g, b)


def kernel(h_sc, h_st, bi_e, bi_graph, sc_e, sc_graph, st_e, st_graph, params):
    Bb, Vsc, Hh = h_sc.shape
    Vst = h_st.shape[1]
    p = params

    names_sc = ["U1", "V1", "W1", "bi_A", "sc_A", "sc_B"]
    names_st = ["U2", "V2", "W2", "bi_B", "st_A", "st_B"]
    w_sc = jnp.stack([p[n]["w"].T for n in names_sc])
    b_sc = jnp.stack([p[n]["b"] for n in names_sc])
    w_st = jnp.stack([p[n]["w"].T for n in names_st])
    b_st = jnp.stack([p[n]["b"] for n in names_st])

    def proj(x, w, b):
        n = x.shape[0] * x.shape[1]
        return pl.pallas_call(
            _proj_kernel,
            out_shape=jax.ShapeDtypeStruct((w.shape[0], n, Hh), jnp.float32),
        )(x.reshape(n, Hh), w, b)

    proj_sc = proj(h_sc, w_sc, b_sc)
    proj_st = proj(h_st, w_st, b_st)
    Uh_sc, Vh_sc, Wh_sc, Abi, Asc, Bsc = (
        proj_sc[k].reshape(Bb, Vsc, Hh) for k in range(6))
    Uh_st, Vh_st, Wh_st, Bbi, Ast, Bst = (
        proj_st[k].reshape(Bb, Vst, Hh) for k in range(6))

    ge = p["ln_e"]["g"][None, :]
    be = p["ln_e"]["b"][None, :]
    gh = p["ln_h"]["g"][None, :]
    bh = p["ln_h"]["b"][None, :]

    bi_out, st2sc, sc2st = _edge_pass(
        bi_e, Abi, Bbi, Vh_st, Vh_sc,
        p["bi_C"]["w"].T, p["bi_C"]["b"][None, :], ge, be)
    sc_out, sc2sc = _edge_pass(
        sc_e, Asc, Bsc, Wh_sc, None,
        p["sc_C"]["w"].T, p["sc_C"]["b"][None, :], ge, be)
    st_out, st2st = _edge_pass(
        st_e, Ast, Bst, Wh_st, None,
        p["st_C"]["w"].T, p["st_C"]["b"][None, :], ge, be)

    h_sc_out = _node_pass(
        h_sc.reshape(Bb * Vsc, Hh), Uh_sc.reshape(Bb * Vsc, Hh),
        st2sc.reshape(Bb * Vsc, Hh), sc2sc.reshape(Bb * Vsc, Hh),
        gh, bh).reshape(Bb, Vsc, Hh)
    h_st_out = _node_pass(
        h_st.reshape(Bb * Vst, Hh), Uh_st.reshape(Bb * Vst, Hh),
        sc2st.reshape(Bb * Vst, Hh), st2st.reshape(Bb * Vst, Hh),
        gh, bh).reshape(Bb, Vst, Hh)

    return (h_sc_out, h_st_out, bi_out, sc_out, st_out)


# speedup vs baseline: 3.3149x; 1.0243x over previous
import jax
import jax.numpy as jnp
from jax.experimental import pallas as pl
from jax.experimental.pallas import tpu as pltpu

_EPS = 1e-5
_BI = 64


def _proj_kernel(x_ref, w_ref, b_ref, o_ref):
    x = x_ref[:]
    for k in range(w_ref.shape[0]):
        o_ref[k] = (jnp.dot(x, w_ref[k], preferred_element_type=jnp.float32)
                    + b_ref[k][None, :])


def _edge_body(e_ref, a_ref, bc_ref, mrow_ref, mcol_ref, cw_ref, cb_ref,
               ge_ref, be_ref, eo_ref, row_ref, col_ref):
    x = e_ref[0]
    xc = jax.lax.dot_general(x, cw_ref[:], (((2,), (0,)), ((), ())),
                             preferred_element_type=jnp.float32)
    e_new = (xc + a_ref[0][:, None, :] + bc_ref[0][None, :, :]
             + cb_ref[:].reshape(1, 1, -1))
    g = jax.nn.sigmoid(e_new)
    row_ref[0] = jnp.sum(g * mrow_ref[0][None, :, :], axis=1)
    if col_ref is not None:
        colc = jnp.sum(g * mcol_ref[0][:, None, :], axis=0)

        @pl.when(pl.program_id(1) == 0)
        def _():
            col_ref[0] = colc

        @pl.when(pl.program_id(1) > 0)
        def _():
            col_ref[0] = col_ref[0] + colc

    mu = jnp.mean(e_new, axis=-1, keepdims=True)
    var = jnp.mean((e_new - mu) ** 2, axis=-1, keepdims=True)
    ln = ((e_new - mu) * jax.lax.rsqrt(var + _EPS)
          * ge_ref[:].reshape(1, 1, -1) + be_ref[:].reshape(1, 1, -1))
    eo_ref[0] = x + jnp.maximum(ln, 0.0)


def _edge_kernel_col(e, a, bc, mr, mc, cw, cb, ge, be, eo, row, col):
    _edge_body(e, a, bc, mr, mc, cw, cb, ge, be, eo, row, col)


def _edge_kernel_nocol(e, a, bc, mr, cw, cb, ge, be, eo, row):
    _edge_body(e, a, bc, mr, None, cw, cb, ge, be, eo, row, None)


def _edge_pass(e, a, bc, mrow, mcol, cw, cb, ge, be):
    Bb, Vi, Vj, Hh = e.shape
    grid = (Bb, Vi // _BI)
    in_specs = [
        pl.BlockSpec((1, _BI, Vj, Hh), lambda b, i: (b, i, 0, 0)),
        pl.BlockSpec((1, _BI, Hh), lambda b, i: (b, i, 0)),
        pl.BlockSpec((1, Vj, Hh), lambda b, i: (b, 0, 0)),
        pl.BlockSpec((1, Vj, Hh), lambda b, i: (b, 0, 0)),
    ]
    if mcol is not None:
        in_specs.append(pl.BlockSpec((1, _BI, Hh), lambda b, i: (b, i, 0)))
    in_specs += [
        pl.BlockSpec((Hh, Hh), lambda b, i: (0, 0)),
        pl.BlockSpec((1, Hh), lambda b, i: (0, 0)),
        pl.BlockSpec((1, Hh), lambda b, i: (0, 0)),
        pl.BlockSpec((1, Hh), lambda b, i: (0, 0)),
    ]
    out_shape = [jax.ShapeDtypeStruct(e.shape, e.dtype),
                 jax.ShapeDtypeStruct((Bb, Vi, Hh), e.dtype)]
    out_specs = [pl.BlockSpec((1, _BI, Vj, Hh), lambda b, i: (b, i, 0, 0)),
                 pl.BlockSpec((1, _BI, Hh), lambda b, i: (b, i, 0))]
    if mcol is not None:
        out_shape.append(jax.ShapeDtypeStruct((Bb, Vj, Hh), e.dtype))
        out_specs.append(pl.BlockSpec((1, Vj, Hh), lambda b, i: (b, 0, 0)))
    fn = _edge_kernel_col if mcol is not None else _edge_kernel_nocol
    args = (e, a, bc, mrow) + ((mcol,) if mcol is not None else ()) + (
        cw, cb, ge, be)
    return pl.pallas_call(
        fn, grid=grid, in_specs=in_specs, out_specs=out_specs,
        out_shape=out_shape,
        compiler_params=pltpu.CompilerParams(
            dimension_semantics=("arbitrary", "arbitrary")),
    )(*args)


def _node_kernel(h_ref, uh_ref, a1_ref, a2_ref, g_ref, b_ref, o_ref):
    s = uh_ref[:] + a1_ref[:] + a2_ref[:]
    mu = jnp.mean(s, axis=-1, keepdims=True)
    var = jnp.mean((s - mu) ** 2, axis=-1, keepdims=True)
    ln = (s - mu) * jax.lax.rsqrt(var + _EPS) * g_ref[:] + b_ref[:]
    o_ref[:] = h_ref[:] + jnp.maximum(ln, 0.0)


def _node_pass(h, uh, a1, a2, g, b):
    n, Hh = h.shape
    return pl.pallas_call(
        _node_kernel,
        out_shape=jax.ShapeDtypeStruct((n, Hh), h.dtype),
    )(h, uh, a1, a2, g, b)


def kernel(h_sc, h_st, bi_e, bi_graph, sc_e, sc_graph, st_e, st_graph, params):
    Bb, Vsc, Hh = h_sc.shape
    Vst = h_st.shape[1]
    p = params

    names_sc = ["U1", "V1", "W1", "bi_A", "sc_A", "sc_B"]
    names_st = ["U2", "V2", "W2", "bi_B", "st_A", "st_B"]
    w_sc = jnp.stack([p[n]["w"].T for n in names_sc])
    b_sc = jnp.stack([p[n]["b"] for n in names_sc])
    w_st = jnp.stack([p[n]["w"].T for n in names_st])
    b_st = jnp.stack([p[n]["b"] for n in names_st])

    def proj(x, w, b):
        n = x.shape[0] * x.shape[1]
        return pl.pallas_call(
            _proj_kernel,
            out_shape=jax.ShapeDtypeStruct((w.shape[0], n, Hh), jnp.float32),
        )(x.reshape(n, Hh), w, b)

    proj_sc = proj(h_sc, w_sc, b_sc)
    proj_st = proj(h_st, w_st, b_st)
    Uh_sc, Vh_sc, Wh_sc, Abi, Asc, Bsc = (
        proj_sc[k].reshape(Bb, Vsc, Hh) for k in range(6))
    Uh_st, Vh_st, Wh_st, Bbi, Ast, Bst = (
        proj_st[k].reshape(Bb, Vst, Hh) for k in range(6))

    ge = p["ln_e"]["g"][None, :]
    be = p["ln_e"]["b"][None, :]
    gh = p["ln_h"]["g"][None, :]
    bh = p["ln_h"]["b"][None, :]

    bi_out, st2sc, sc2st = _edge_pass(
        bi_e, Abi, Bbi, Vh_st, Vh_sc,
        p["bi_C"]["w"].T, p["bi_C"]["b"][None, :], ge, be)
    sc_out, sc2sc = _edge_pass(
        sc_e, Asc, Bsc, Wh_sc, None,
        p["sc_C"]["w"].T, p["sc_C"]["b"][None, :], ge, be)
    st_out, st2st = _edge_pass(
        st_e, Ast, Bst, Wh_st, None,
        p["st_C"]["w"].T, p["st_C"]["b"][None, :], ge, be)

    h_sc_out = _node_pass(
        h_sc.reshape(Bb * Vsc, Hh), Uh_sc.reshape(Bb * Vsc, Hh),
        st2sc.reshape(Bb * Vsc, Hh), sc2sc.reshape(Bb * Vsc, Hh),
        gh, bh).reshape(Bb, Vsc, Hh)
    h_st_out = _node_pass(
        h_st.reshape(Bb * Vst, Hh), Uh_st.reshape(Bb * Vst, Hh),
        sc2st.reshape(Bb * Vst, Hh), st2st.reshape(Bb * Vst, Hh),
        gh, bh).reshape(Bb, Vst, Hh)

    return (h_sc_out, h_st_out, bi_out, sc_out, st_out)
